# baseline (device time: 33099 ns/iter reference)
import jax
import jax.numpy as jnp
from jax import lax
from jax.experimental import pallas as pl
from jax.experimental.pallas import tpu as pltpu

N_DEV = 4
B, SQ, SKV = 2, 256, 256
HQ_LOCAL, DH = 4, 64
D_MODEL = 512
WINDOW = 128


def kernel(x, Wq, K_ext, V_ext, Wo):
    my = lax.axis_index("i")
    K_loc = lax.dynamic_slice_in_dim(K_ext, my * HQ_LOCAL, HQ_LOCAL, axis=2)
    V_loc = lax.dynamic_slice_in_dim(V_ext, my * HQ_LOCAL, HQ_LOCAL, axis=2)
    K_loc = jnp.transpose(K_loc, (0, 2, 1, 3)).astype(jnp.bfloat16)
    V_loc = jnp.transpose(V_loc, (0, 2, 1, 3)).astype(jnp.bfloat16)
    xb = x.astype(jnp.bfloat16)
    Wqb = Wq.astype(jnp.bfloat16)
    Wob = Wo.astype(jnp.bfloat16)

    def body(x_ref, wq_ref, k_ref, v_ref, wo_ref, out_ref,
             comm_ref, send_sems, recv_sems):
        my_pos = lax.axis_index("i")
        left = lax.rem(my_pos + (N_DEV - 1), N_DEV)
        right = lax.rem(my_pos + 1, N_DEV)

        barrier_sem = pltpu.get_barrier_semaphore()
        for nbr in (left, right):
            pl.semaphore_signal(
                barrier_sem, inc=1,
                device_id=(nbr,), device_id_type=pl.DeviceIdType.MESH,
            )
        pl.semaphore_wait(barrier_sem, 2)

        qi = lax.broadcasted_iota(jnp.int32, (SQ, SKV), 0)
        ki = lax.broadcasted_iota(jnp.int32, (SQ, SKV), 1)
        mask = jnp.abs(qi - ki) <= WINDOW

        for b in range(B):
            q_b = jnp.dot(
                x_ref[b], wq_ref[...], preferred_element_type=jnp.float32
            ).astype(jnp.bfloat16)
            ctx_cols = []
            for h in range(HQ_LOCAL):
                q_bh = q_b[:, h * DH:(h + 1) * DH]
                k_bh = k_ref[b, h]
                v_bh = v_ref[b, h]
                s = lax.dot_general(
                    q_bh, k_bh,
                    dimension_numbers=(((1,), (1,)), ((), ())),
                    preferred_element_type=jnp.float32,
                ) * 0.125
                s = jnp.where(mask, s, -1e9)
                s = s - jnp.max(s, axis=-1, keepdims=True)
                w = jnp.exp(s)
                w = w / jnp.sum(w, axis=-1, keepdims=True)
                ctx_bh = jnp.dot(
                    w.astype(jnp.bfloat16), v_bh,
                    preferred_element_type=jnp.float32,
                )
                ctx_cols.append(ctx_bh.astype(jnp.bfloat16))
            ctx_b = jnp.concatenate(ctx_cols, axis=1)
            part_b = jnp.dot(
                ctx_b, wo_ref[...], preferred_element_type=jnp.float32
            )
            out_ref[b, :, :] = part_b
            comm_ref[0, b, :, :] = part_b.astype(jnp.bfloat16)

        for h in range(N_DEV - 1):
            rdma = pltpu.make_async_remote_copy(
                src_ref=comm_ref.at[h],
                dst_ref=comm_ref.at[h + 1],
                send_sem=send_sems.at[h],
                recv_sem=recv_sems.at[h],
                device_id=(right,),
                device_id_type=pl.DeviceIdType.MESH,
            )
            rdma.start()
            rdma.wait()
            out_ref[...] = out_ref[...] + comm_ref[h + 1].astype(jnp.float32)

    return pl.pallas_call(
        body,
        out_shape=jax.ShapeDtypeStruct((B, SQ, D_MODEL), jnp.float32),
        in_specs=[pl.BlockSpec(memory_space=pltpu.VMEM)] * 5,
        out_specs=pl.BlockSpec(memory_space=pltpu.VMEM),
        scratch_shapes=[
            pltpu.VMEM((N_DEV, B, SQ, D_MODEL), jnp.bfloat16),
            pltpu.SemaphoreType.DMA((N_DEV - 1,)),
            pltpu.SemaphoreType.DMA((N_DEV - 1,)),
        ],
        compiler_params=pltpu.CompilerParams(collective_id=0),
    )(xb, Wqb, K_loc, V_loc, Wob)


# device time: 20245 ns/iter; 1.6349x vs baseline; 1.6349x over previous
import jax
import jax.numpy as jnp
from jax import lax
from jax.experimental import pallas as pl
from jax.experimental.pallas import tpu as pltpu

N_DEV = 4
B, SQ, SKV = 2, 256, 256
HQ_LOCAL, DH = 4, 64
D_MODEL = 512
WINDOW = 128
QROWS = SQ // N_DEV


def kernel(x, Wq, K_ext, V_ext, Wo):
    my = lax.axis_index("i")
    K_loc = lax.dynamic_slice_in_dim(K_ext, my * HQ_LOCAL, HQ_LOCAL, axis=2)
    V_loc = lax.dynamic_slice_in_dim(V_ext, my * HQ_LOCAL, HQ_LOCAL, axis=2)
    K_loc = jnp.transpose(K_loc, (0, 2, 1, 3)).astype(jnp.bfloat16)
    V_loc = jnp.transpose(V_loc, (0, 2, 1, 3)).astype(jnp.bfloat16)
    xb = x.astype(jnp.bfloat16)
    Wqb = Wq.astype(jnp.bfloat16)
    Wob = Wo.astype(jnp.bfloat16)

    def body(x_ref, wq_ref, k_ref, v_ref, wo_ref, out_ref,
             part_ref, rs_recv, ag_send, ag_recv,
             rs_send_sems, rs_recv_sems, ag_send_sems, ag_recv_sems):
        my_pos = lax.axis_index("i")

        barrier_sem = pltpu.get_barrier_semaphore()
        for k in (1, 2, 3):
            pl.semaphore_signal(
                barrier_sem, inc=1,
                device_id=(lax.rem(my_pos + k, N_DEV),),
                device_id_type=pl.DeviceIdType.MESH,
            )
        pl.semaphore_wait(barrier_sem, 3)

        qi = lax.broadcasted_iota(jnp.int32, (SQ, SKV), 0)
        ki = lax.broadcasted_iota(jnp.int32, (SQ, SKV), 1)
        mask = jnp.abs(qi - ki) <= WINDOW

        x2 = jnp.reshape(x_ref[...], (B * SQ, x_ref.shape[-1]))
        q_all = jnp.dot(
            x2, wq_ref[...], preferred_element_type=jnp.float32
        ).astype(jnp.bfloat16)
        ctx_rows = []
        for b in range(B):
            ctx_cols = []
            for h in range(HQ_LOCAL):
                q_bh = q_all[b * SQ:(b + 1) * SQ, h * DH:(h + 1) * DH]
                k_bh = k_ref[b, h]
                v_bh = v_ref[b, h]
                s = lax.dot_general(
                    q_bh, k_bh,
                    dimension_numbers=(((1,), (1,)), ((), ())),
                    preferred_element_type=jnp.float32,
                ) * 0.125
                s = jnp.where(mask, s, -1e9)
                s = s - jnp.max(s, axis=-1, keepdims=True)
                w = jnp.exp(s)
                w = w / jnp.sum(w, axis=-1, keepdims=True)
                ctx_bh = jnp.dot(
                    w.astype(jnp.bfloat16), v_bh,
                    preferred_element_type=jnp.float32,
                )
                ctx_cols.append(ctx_bh.astype(jnp.bfloat16))
            ctx_rows.append(jnp.concatenate(ctx_cols, axis=1))
        ctx_all = jnp.concatenate(ctx_rows, axis=0)
        part2 = jnp.dot(
            ctx_all, wo_ref[...], preferred_element_type=jnp.float32
        )
        partial = jnp.reshape(part2, (B, SQ, D_MODEL))
        out_ref[...] = partial
        part_ref[...] = partial.astype(jnp.bfloat16)

        rs = {}
        for k in (1, 2, 3):
            t = lax.rem(my_pos + k, N_DEV)
            rs[k] = pltpu.make_async_remote_copy(
                src_ref=part_ref.at[:, pl.ds(t * QROWS, QROWS), :],
                dst_ref=rs_recv.at[3 - k],
                send_sem=rs_send_sems.at[k - 1],
                recv_sem=rs_recv_sems.at[3 - k],
                device_id=(t,),
                device_id_type=pl.DeviceIdType.MESH,
            )
            rs[k].start()

        acc = out_ref[:, pl.ds(my_pos * QROWS, QROWS), :]
        for k in (3, 1, 2):
            rs[k].wait_recv()
            acc = acc + rs_recv[3 - k].astype(jnp.float32)
        out_ref[:, pl.ds(my_pos * QROWS, QROWS), :] = acc
        ag_send[...] = acc.astype(jnp.bfloat16)

        ag = {}
        for k in (1, 2, 3):
            t = lax.rem(my_pos + k, N_DEV)
            ag[k] = pltpu.make_async_remote_copy(
                src_ref=ag_send,
                dst_ref=ag_recv.at[3 - k],
                send_sem=ag_send_sems.at[k - 1],
                recv_sem=ag_recv_sems.at[3 - k],
                device_id=(t,),
                device_id_type=pl.DeviceIdType.MESH,
            )
            ag[k].start()
        for k in (3, 1, 2):
            ag[k].wait_recv()
            sender = lax.rem(my_pos + (N_DEV - k), N_DEV)
            out_ref[:, pl.ds(sender * QROWS, QROWS), :] = (
                ag_recv[3 - k].astype(jnp.float32)
            )

        for k in (1, 2, 3):
            rs[k].wait_send()
            ag[k].wait_send()

    return pl.pallas_call(
        body,
        out_shape=jax.ShapeDtypeStruct((B, SQ, D_MODEL), jnp.float32),
        in_specs=[pl.BlockSpec(memory_space=pltpu.VMEM)] * 5,
        out_specs=pl.BlockSpec(memory_space=pltpu.VMEM),
        scratch_shapes=[
            pltpu.VMEM((B, SQ, D_MODEL), jnp.bfloat16),
            pltpu.VMEM((3, B, QROWS, D_MODEL), jnp.bfloat16),
            pltpu.VMEM((B, QROWS, D_MODEL), jnp.bfloat16),
            pltpu.VMEM((3, B, QROWS, D_MODEL), jnp.bfloat16),
            pltpu.SemaphoreType.DMA((3,)),
            pltpu.SemaphoreType.DMA((3,)),
            pltpu.SemaphoreType.DMA((3,)),
            pltpu.SemaphoreType.DMA((3,)),
        ],
        compiler_params=pltpu.CompilerParams(collective_id=0),
    )(xb, Wqb, K_loc, V_loc, Wob)
